# baseline (speedup 1.0000x reference)
import jax
import jax.numpy as jnp
from jax.experimental import pallas as pl

_DIM = 32768
_C = 4
_BS = 8
_NSTEPS = 10
_INV_TEMP = 0.5
_INV_STEP = 5.0


def _pick_sub(p4, idx1):
    cls = jax.lax.broadcasted_iota(jnp.int32, (_C, _DIM), 0)
    return jnp.sum(jnp.where(cls == idx1, p4, 0.0), axis=0, keepdims=True)


def _log_softmax_sub(l4):
    m = jnp.max(l4, axis=0, keepdims=True)
    sh = l4 - m
    se = jnp.sum(jnp.exp(sh), axis=0, keepdims=True)
    return sh - jnp.log(se)


def _pick_rows(p4, idx):
    return jnp.where(
        idx == 0, p4[0:1],
        jnp.where(idx == 1, p4[1:2],
                  jnp.where(idx == 2, p4[2:3], p4[3:4])))


def _step_kernel(gum_ref, u_ref, wp_ref, x_ref, out_ref):
    i = pl.program_id(0)

    @pl.when(i == 0)
    def _():
        out_ref[...] = x_ref[...]

    xc = out_ref[...]
    xc0 = xc[0:1, :]
    row0 = jax.lax.broadcasted_iota(jnp.int32, (_BS, 1), 0) == 0
    cls = jax.lax.broadcasted_iota(jnp.int32, (_C, _DIM), 0)

    W4 = wp_ref[0:_C, :]
    G4 = W4 * _INV_TEMP

    Gc0 = _pick_sub(G4, xc0)
    first4 = G4 - Gc0
    lo4 = first4 - _INV_STEP
    oh04 = cls == xc0
    logits_r04 = jnp.where(oh04, first4, lo4)

    xd = jnp.zeros((_BS, _DIM), jnp.int32)
    best = None
    for c in range(_C):
        gc = gum_ref[0, c]
        tc = jnp.where(row0 & oh04[c:c + 1],
                       first4[c:c + 1] + gc[0:1, :],
                       lo4[c:c + 1] + gc)
        if best is None:
            best = tc
        else:
            upd = tc > best
            xd = jnp.where(upd, c, xd)
            best = jnp.where(upd, tc, best)

    logp_sh4 = _log_softmax_sub(lo4)
    logp_r04 = _log_softmax_sub(logits_r04)
    xd0 = xd[0:1, :]
    pf = jnp.where(row0,
                   _pick_sub(logp_r04, xd0),
                   _pick_rows(logp_sh4, xd))
    lp_fwd = jnp.sum(pf, axis=1, keepdims=True)

    Gd0 = _pick_sub(G4, xd0)
    first_d4 = G4 - Gd0
    lod4 = first_d4 - _INV_STEP
    ohd04 = cls == xd0
    logp_dsh4 = _log_softmax_sub(lod4)
    logp_dr04 = _log_softmax_sub(jnp.where(ohd04, first_d4, lod4))
    pr = jnp.where(row0,
                   _pick_sub(logp_dr04, xc0),
                   _pick_rows(logp_dsh4, xc))
    lp_rev = jnp.sum(pr, axis=1, keepdims=True)

    e_d = jnp.sum(_pick_sub(W4, xd0), axis=1, keepdims=True)
    e_c = jnp.sum(_pick_sub(W4, xc0), axis=1, keepdims=True)
    m_term = e_d - e_c

    la = (m_term + lp_rev) - lp_fwd
    acc = jnp.exp(la) > u_ref[0]
    out_ref[...] = jnp.where(acc, xd, xc)


def kernel(x, W):
    xdtype = x.dtype
    xi = x.astype(jnp.int32)

    key = jax.random.key(42)
    gums, us = [], []
    for _ in range(_NSTEPS):
        key, ks, kr = jax.random.split(key, 3)
        gums.append(jax.random.gumbel(ks, (_BS, _DIM, _C), jnp.float32))
        us.append(jax.random.uniform(kr, (_BS,)))
    gum = jnp.stack(gums).transpose(0, 3, 1, 2)
    u = jnp.stack(us).reshape(_NSTEPS, _BS, 1)
    wp = jnp.concatenate([W.T, jnp.zeros((4, _DIM), jnp.float32)], axis=0)

    out = pl.pallas_call(
        _step_kernel,
        grid=(_NSTEPS,),
        in_specs=[
            pl.BlockSpec((1, _C, _BS, _DIM), lambda i: (i, 0, 0, 0)),
            pl.BlockSpec((1, _BS, 1), lambda i: (i, 0, 0)),
            pl.BlockSpec((8, _DIM), lambda i: (0, 0)),
            pl.BlockSpec((_BS, _DIM), lambda i: (0, 0)),
        ],
        out_specs=pl.BlockSpec((_BS, _DIM), lambda i: (0, 0)),
        out_shape=jax.ShapeDtypeStruct((_BS, _DIM), jnp.int32),
    )(gum, u, wp, xi)
    return out.astype(xdtype)
